# baseline (speedup 1.0000x reference)
import functools

import jax
import jax.numpy as jnp
from jax import lax
from jax.experimental import pallas as pl
from jax.experimental.pallas import tpu as pltpu
from jax.experimental.pallas import tpu_sc as plsc

EMB_DIM = 32
CHUNK = 128


@functools.lru_cache(maxsize=None)
def _make_gather(batch):
    info = plsc.get_sparse_core_info()
    num_cores, num_subcores = info.num_cores, info.num_subcores
    num_workers = num_cores * num_subcores
    per_worker = batch // num_workers
    steps = per_worker // CHUNK
    assert per_worker % CHUNK == 0 and batch % num_workers == 0

    mesh = plsc.VectorSubcoreMesh(core_axis_name="c", subcore_axis_name="s")

    @functools.partial(
        pl.kernel,
        mesh=mesh,
        compiler_params=pltpu.CompilerParams(use_tc_tiling_on_sc=False),
        out_type=jax.ShapeDtypeStruct((batch, EMB_DIM), jnp.float32),
        scratch_types=[
            pltpu.VMEM((CHUNK,), jnp.int32),
            pltpu.VMEM((CHUNK, EMB_DIM), jnp.float32),
            pltpu.SemaphoreType.DMA,
        ],
    )
    def gather_kernel(table_hbm, idx_hbm, out_hbm, idx_v, rows_v, sem):
        wid = lax.axis_index("s") * num_cores + lax.axis_index("c")
        base = wid * per_worker

        def body(i, carry):
            off = base + i * CHUNK
            pltpu.sync_copy(idx_hbm.at[pl.ds(off, CHUNK)], idx_v)
            pltpu.async_copy(table_hbm.at[idx_v], rows_v, sem).wait()
            pltpu.sync_copy(rows_v, out_hbm.at[pl.ds(off, CHUNK)])
            return carry

        lax.fori_loop(0, steps, body, 0)

    return gather_kernel


@jax.jit
def kernel(x, weight):
    b, h = x.shape
    idx = x.reshape(b * h)
    out = _make_gather(b * h)(weight, idx)
    return out.reshape(b, h, EMB_DIM)


# speedup vs baseline: 1.1368x; 1.1368x over previous
import functools

import jax
import jax.numpy as jnp
from jax import lax
from jax.experimental import pallas as pl
from jax.experimental.pallas import tpu as pltpu
from jax.experimental.pallas import tpu_sc as plsc

EMB_DIM = 32
CHUNK = 128
NBUF = 4


@functools.lru_cache(maxsize=None)
def _make_gather(batch):
    info = plsc.get_sparse_core_info()
    num_cores, num_subcores = info.num_cores, info.num_subcores
    num_workers = num_cores * num_subcores
    per_worker = batch // num_workers
    steps = per_worker // CHUNK
    groups = steps // NBUF
    assert batch % num_workers == 0 and per_worker % (CHUNK * NBUF) == 0

    mesh = plsc.VectorSubcoreMesh(core_axis_name="c", subcore_axis_name="s")

    @functools.partial(
        pl.kernel,
        mesh=mesh,
        compiler_params=pltpu.CompilerParams(use_tc_tiling_on_sc=False),
        out_type=jax.ShapeDtypeStruct((batch, EMB_DIM), jnp.float32),
        scratch_types=(
            [pltpu.VMEM((per_worker,), jnp.int32)]
            + [pltpu.VMEM((CHUNK, EMB_DIM), jnp.float32) for _ in range(NBUF)]
            + [pltpu.SemaphoreType.DMA for _ in range(2 * NBUF)]
        ),
    )
    def gather_kernel(table_hbm, idx_hbm, out_hbm, idx_v, *rest):
        rows = rest[:NBUF]
        gsem = rest[NBUF : 2 * NBUF]
        osem = rest[2 * NBUF :]
        wid = lax.axis_index("s") * num_cores + lax.axis_index("c")
        base = wid * per_worker

        pltpu.sync_copy(idx_hbm.at[pl.ds(base, per_worker)], idx_v)

        def start_gather(step, b):
            pltpu.async_copy(
                table_hbm.at[idx_v.at[pl.ds(step * CHUNK, CHUNK)]],
                rows[b],
                gsem[b],
            )

        def wait_gather(step, b):
            pltpu.make_async_copy(
                table_hbm.at[idx_v.at[pl.ds(step * CHUNK, CHUNK)]],
                rows[b],
                gsem[b],
            ).wait()

        def out_slice(step):
            return out_hbm.at[pl.ds(base + step * CHUNK, CHUNK)]

        for b in range(NBUF):
            start_gather(b, b)

        def body(g, carry):
            for b in range(NBUF):
                step = g * NBUF + b
                wait_gather(step, b)
                pltpu.async_copy(rows[b], out_slice(step), osem[b])
                pltpu.make_async_copy(rows[b], out_slice(step), osem[b]).wait()
                start_gather(step + NBUF, b)
            return carry

        lax.fori_loop(0, groups - 1, body, 0)

        for b in range(NBUF):
            step = (groups - 1) * NBUF + b
            wait_gather(step, b)
            pltpu.async_copy(rows[b], out_slice(step), osem[b])
        for b in range(NBUF):
            step = (groups - 1) * NBUF + b
            pltpu.make_async_copy(rows[b], out_slice(step), osem[b]).wait()

    return gather_kernel


@jax.jit
def kernel(x, weight):
    b, h = x.shape
    idx = x.reshape(b * h)
    out = _make_gather(b * h)(weight, idx)
    return out.reshape(b, h, EMB_DIM)


# speedup vs baseline: 1.6059x; 1.4127x over previous
import functools

import jax
import jax.numpy as jnp
from jax import lax
from jax.experimental import pallas as pl
from jax.experimental.pallas import tpu as pltpu
from jax.experimental.pallas import tpu_sc as plsc

EMB_DIM = 32
BLK = 128


@functools.lru_cache(maxsize=None)
def _make_lookup(batch, hist):
    info = plsc.get_sparse_core_info()
    num_cores, num_subcores = info.num_cores, info.num_subcores
    num_workers = num_cores * num_subcores
    b_per_w = batch // num_workers
    nblk = b_per_w // BLK
    steps = hist * nblk
    assert batch % num_workers == 0 and b_per_w % BLK == 0 and steps % 2 == 0

    mesh = plsc.VectorSubcoreMesh(core_axis_name="c", subcore_axis_name="s")

    @functools.partial(
        pl.kernel,
        mesh=mesh,
        compiler_params=pltpu.CompilerParams(needs_layout_passes=False),
        out_type=jax.ShapeDtypeStruct((hist, EMB_DIM, batch), jnp.float32),
        scratch_types=(
            [pltpu.VMEM((hist, b_per_w), jnp.int32)]
            + [pltpu.VMEM((BLK,), jnp.int32) for _ in range(2)]
            + [pltpu.VMEM((BLK,), jnp.int32) for _ in range(2)]
            + [pltpu.VMEM((BLK, 128), jnp.float32) for _ in range(2)]
            + [pltpu.VMEM((1, EMB_DIM, BLK), jnp.float32) for _ in range(2)]
            + [pltpu.SemaphoreType.DMA for _ in range(4)]
        ),
    )
    def lookup(w4_hbm, xt_hbm, out_hbm, idx_all, q0, q1, r0, r1, g0, g1, t0, t1,
               sg0, sg1, so0, so1):
        qs, rs, gs, ts = (q0, q1), (r0, r1), (g0, g1), (t0, t1)
        sgs, sos = (sg0, sg1), (so0, so1)
        wid = lax.axis_index("s") * num_cores + lax.axis_index("c")
        bbase = wid * b_per_w

        pltpu.sync_copy(xt_hbm.at[:, pl.ds(bbase, b_per_w)], idx_all)

        def prep(s, buf):
            h = s // nblk
            off = (s % nblk) * BLK
            for grp in range(BLK // 16):
                v = idx_all[h, pl.ds(off + grp * 16, 16)]
                qs[buf][pl.ds(grp * 16, 16)] = v >> 2
                rs[buf][pl.ds(grp * 16, 16)] = (v & 3) * 32

        def fire_gather(buf):
            pltpu.async_copy(w4_hbm.at[qs[buf]], gs[buf], sgs[buf])

        def wait_gather(buf):
            pltpu.make_async_copy(w4_hbm.at[qs[buf]], gs[buf], sgs[buf]).wait()

        def out_slice(s):
            h = s // nblk
            b0 = bbase + (s % nblk) * BLK
            return out_hbm.at[pl.ds(h, 1), :, pl.ds(b0, BLK)]

        def transpose_select(buf):
            g, t, r = gs[buf], ts[buf], rs[buf]
            for grp in range(BLK // 16):
                jvec = lax.iota(jnp.int32, 16) + (grp * 16)
                rvec = r[pl.ds(grp * 16, 16)]
                for d in range(EMB_DIM):
                    t[0, d, pl.ds(grp * 16, 16)] = plsc.load_gather(
                        g, [jvec, rvec + d]
                    )

        def step(s, buf, fire_next, drain_prev):
            if fire_next:
                prep(s + 1, 1 - buf)
                fire_gather(1 - buf)
            wait_gather(buf)
            if drain_prev:

                @pl.when(s >= 2)
                def _():
                    pltpu.make_async_copy(ts[buf], out_slice(s - 2), sos[buf]).wait()

            transpose_select(buf)
            pltpu.async_copy(ts[buf], out_slice(s), sos[buf])

        prep(0, 0)
        fire_gather(0)

        def pair(g2, carry):
            s0 = 2 * g2
            step(s0, 0, True, True)
            step(s0 + 1, 1, True, True)
            return carry

        lax.fori_loop(0, steps // 2 - 1, pair, 0)

        s0 = steps - 2
        step(s0, 0, True, True)
        step(s0 + 1, 1, False, True)
        pltpu.make_async_copy(ts[0], out_slice(s0), sos[0]).wait()
        pltpu.make_async_copy(ts[1], out_slice(s0 + 1), sos[1]).wait()

    return lookup


@jax.jit
def kernel(x, weight):
    b, h = x.shape
    w4 = weight.reshape(weight.shape[0] // 4, 4 * EMB_DIM)
    xt = x.T
    out2 = _make_lookup(b, h)(w4, xt)
    return jnp.transpose(out2, (2, 0, 1))


# speedup vs baseline: 1.9710x; 1.2273x over previous
import functools

import jax
import jax.numpy as jnp
from jax import lax
from jax.experimental import pallas as pl
from jax.experimental.pallas import tpu as pltpu
from jax.experimental.pallas import tpu_sc as plsc

EMB_DIM = 32
BLK = 128


@functools.lru_cache(maxsize=None)
def _make_lookup(batch, hist):
    info = plsc.get_sparse_core_info()
    num_cores, num_subcores = info.num_cores, info.num_subcores
    num_workers = num_cores * num_subcores
    b_per_w = batch // num_workers
    nblk = b_per_w // BLK
    steps = hist * nblk
    assert batch % num_workers == 0 and b_per_w % BLK == 0 and steps % 2 == 0

    mesh = plsc.VectorSubcoreMesh(core_axis_name="c", subcore_axis_name="s")

    @functools.partial(
        pl.kernel,
        mesh=mesh,
        compiler_params=pltpu.CompilerParams(
            use_tc_tiling_on_sc=False, needs_layout_passes=False
        ),
        out_type=jax.ShapeDtypeStruct((hist * batch, EMB_DIM), jnp.float32),
        scratch_types=(
            [pltpu.VMEM((hist, b_per_w), jnp.int32)]
            + [pltpu.VMEM((BLK, EMB_DIM), jnp.float32) for _ in range(4)]
            + [pltpu.SemaphoreType.DMA]
            + [pltpu.SemaphoreType.DMA for _ in range(4)]
            + [pltpu.SemaphoreType.DMA for _ in range(4)]
        ),
    )
    def lookup(w_hbm, xf_hbm, out_hbm, idx_all, g0, g1, g2, g3, si,
               sg0, sg1, sg2, sg3, so0, so1, so2, so3):
        gs, sgs, sos = (g0, g1, g2, g3), (sg0, sg1, sg2, sg3), (so0, so1, so2, so3)
        wid = lax.axis_index("s") * num_cores + lax.axis_index("c")
        bbase = wid * b_per_w

        for h in range(hist):
            pltpu.async_copy(
                xf_hbm.at[pl.ds(h * batch + bbase, b_per_w)], idx_all.at[h], si
            )
        for h in range(hist):
            pltpu.make_async_copy(
                xf_hbm.at[pl.ds(h * batch + bbase, b_per_w)], idx_all.at[h], si
            ).wait()

        def idx_slice(s):
            h = s // nblk
            off = (s % nblk) * BLK
            return idx_all.at[h, pl.ds(off, BLK)]

        def out_slice(s):
            h = s // nblk
            row0 = h * batch + bbase + (s % nblk) * BLK
            return out_hbm.at[pl.ds(row0, BLK)]

        def fire_gather(s, buf):
            pltpu.async_copy(w_hbm.at[idx_slice(s)], gs[buf], sgs[buf])

        def wait_gather(s, buf):
            pltpu.make_async_copy(w_hbm.at[idx_slice(s)], gs[buf], sgs[buf]).wait()

        def drain_out(s, buf):
            pltpu.make_async_copy(gs[buf], out_slice(s), sos[buf]).wait()

        def step(s, k, fire_next):
            wait_gather(s, k)
            if fire_next:
                nslot = (k + 3) % 4
                if isinstance(s, int):
                    if s >= 1:
                        drain_out(s - 1, nslot)
                else:

                    @pl.when(s >= 1)
                    def _():
                        drain_out(s - 1, nslot)

                fire_gather(s + 3, nslot)
            pltpu.async_copy(gs[k], out_slice(s), sos[k])

        for k in range(3):
            fire_gather(k, k)

        def quad(g4, carry):
            for k in range(4):
                s = 4 * g4 + k
                step(s, k, True)
            return carry

        lax.fori_loop(0, steps // 4 - 1, quad, 0)

        for k in range(4):
            s = steps - 4 + k
            step(s, k, s + 3 < steps)
        for k in range(4):
            s = steps - 4 + k
            drain_out(s, s % 4)

    return lookup


@jax.jit
def kernel(x, weight):
    b, h = x.shape
    xf = x.T.reshape(b * h)
    out = _make_lookup(b, h)(weight, xf)
    return jnp.transpose(out.reshape(h, b, EMB_DIM), (1, 0, 2))


# speedup vs baseline: 1.9715x; 1.0003x over previous
import functools

import jax
import jax.numpy as jnp
from jax import lax
from jax.experimental import pallas as pl
from jax.experimental.pallas import tpu as pltpu
from jax.experimental.pallas import tpu_sc as plsc

EMB_DIM = 32
BLK = 128


@functools.lru_cache(maxsize=None)
def _make_lookup(batch, hist):
    info = plsc.get_sparse_core_info()
    num_cores, num_subcores = info.num_cores, info.num_subcores
    num_workers = num_cores * num_subcores
    b_per_w = batch // num_workers
    nblk = b_per_w // BLK
    steps = hist * nblk
    assert batch % num_workers == 0 and b_per_w % BLK == 0 and steps % 2 == 0

    mesh = plsc.VectorSubcoreMesh(core_axis_name="c", subcore_axis_name="s")

    @functools.partial(
        pl.kernel,
        mesh=mesh,
        compiler_params=pltpu.CompilerParams(
            use_tc_tiling_on_sc=False, needs_layout_passes=False
        ),
        out_type=jax.ShapeDtypeStruct((hist, batch, EMB_DIM), jnp.float32),
        scratch_types=(
            [pltpu.VMEM((hist, b_per_w), jnp.int32)]
            + [pltpu.VMEM((BLK, EMB_DIM), jnp.float32) for _ in range(4)]
            + [pltpu.SemaphoreType.DMA]
            + [pltpu.SemaphoreType.DMA for _ in range(4)]
            + [pltpu.SemaphoreType.DMA for _ in range(4)]
        ),
    )
    def lookup(w_hbm, xf_hbm, out_hbm, idx_all, g0, g1, g2, g3, si,
               sg0, sg1, sg2, sg3, so0, so1, so2, so3):
        gs, sgs, sos = (g0, g1, g2, g3), (sg0, sg1, sg2, sg3), (so0, so1, so2, so3)
        wid = lax.axis_index("s") * num_cores + lax.axis_index("c")
        bbase = wid * b_per_w

        for h in range(hist):
            pltpu.async_copy(
                xf_hbm.at[pl.ds(h * batch + bbase, b_per_w)], idx_all.at[h], si
            )
        for h in range(hist):
            pltpu.make_async_copy(
                xf_hbm.at[pl.ds(h * batch + bbase, b_per_w)], idx_all.at[h], si
            ).wait()

        def idx_slice(s):
            h = s // nblk
            off = (s % nblk) * BLK
            return idx_all.at[h, pl.ds(off, BLK)]

        def out_slice(s):
            h = s // nblk
            b0 = bbase + (s % nblk) * BLK
            return out_hbm.at[h, pl.ds(b0, BLK), :]

        def fire_gather(s, buf):
            pltpu.async_copy(w_hbm.at[idx_slice(s)], gs[buf], sgs[buf])

        def wait_gather(s, buf):
            pltpu.make_async_copy(w_hbm.at[idx_slice(s)], gs[buf], sgs[buf]).wait()

        def drain_out(s, buf):
            pltpu.make_async_copy(gs[buf], out_slice(s), sos[buf]).wait()

        def step(s, k, fire_next):
            wait_gather(s, k)
            if fire_next:
                nslot = (k + 3) % 4
                if isinstance(s, int):
                    if s >= 1:
                        drain_out(s - 1, nslot)
                else:

                    @pl.when(s >= 1)
                    def _():
                        drain_out(s - 1, nslot)

                fire_gather(s + 3, nslot)
            pltpu.async_copy(gs[k], out_slice(s), sos[k])

        for k in range(3):
            fire_gather(k, k)

        def quad(g4, carry):
            for k in range(4):
                s = 4 * g4 + k
                step(s, k, True)
            return carry

        lax.fori_loop(0, steps // 4 - 1, quad, 0)

        for k in range(4):
            s = steps - 4 + k
            step(s, k, s + 3 < steps)
        for k in range(4):
            s = steps - 4 + k
            drain_out(s, s % 4)

    return lookup


@jax.jit
def kernel(x, weight):
    b, h = x.shape
    xf = x.T.reshape(b * h)
    out = _make_lookup(b, h)(weight, xf)
    return jnp.transpose(out, (1, 0, 2))


# speedup vs baseline: 1.9721x; 1.0003x over previous
import functools

import jax
import jax.numpy as jnp
from jax import lax
from jax.experimental import pallas as pl
from jax.experimental.pallas import tpu as pltpu
from jax.experimental.pallas import tpu_sc as plsc

EMB_DIM = 32
BLK = 128


@functools.lru_cache(maxsize=None)
def _make_lookup(batch, hist):
    info = plsc.get_sparse_core_info()
    num_cores, num_subcores = info.num_cores, info.num_subcores
    num_workers = num_cores * num_subcores
    b_per_w = batch // num_workers
    nblk = b_per_w // BLK
    steps = hist * nblk
    assert batch % num_workers == 0 and b_per_w % BLK == 0 and steps % 4 == 0

    mesh = plsc.VectorSubcoreMesh(core_axis_name="c", subcore_axis_name="s")

    @functools.partial(
        pl.kernel,
        mesh=mesh,
        compiler_params=pltpu.CompilerParams(
            use_tc_tiling_on_sc=False, needs_layout_passes=False
        ),
        out_type=jax.ShapeDtypeStruct((hist, batch, EMB_DIM), jnp.float32),
        scratch_types=(
            [pltpu.VMEM((hist, b_per_w), jnp.int32)]
            + [pltpu.VMEM((BLK, EMB_DIM), jnp.float32) for _ in range(4)]
            + [pltpu.SemaphoreType.DMA]
            + [pltpu.SemaphoreType.DMA for _ in range(4)]
            + [pltpu.SemaphoreType.DMA for _ in range(4)]
        ),
    )
    def lookup(w_hbm, xf_hbm, out_hbm, idx_all, g0, g1, g2, g3, si,
               sg0, sg1, sg2, sg3, so0, so1, so2, so3):
        gs, sgs, sos = (g0, g1, g2, g3), (sg0, sg1, sg2, sg3), (so0, so1, so2, so3)
        wid = lax.axis_index("s") * num_cores + lax.axis_index("c")
        bbase = wid * b_per_w

        for h in range(hist):
            pltpu.async_copy(
                xf_hbm.at[pl.ds(h * batch + bbase, b_per_w)], idx_all.at[h], si
            )
        for h in range(hist):
            pltpu.make_async_copy(
                xf_hbm.at[pl.ds(h * batch + bbase, b_per_w)], idx_all.at[h], si
            ).wait()

        def idx_slice(s):
            h = s // nblk
            off = (s % nblk) * BLK
            return idx_all.at[h, pl.ds(off, BLK)]

        def out_slice(s):
            h = s // nblk
            b0 = bbase + (s % nblk) * BLK
            return out_hbm.at[h, pl.ds(b0, BLK), :]

        def fire_gather(s, buf):
            pltpu.async_copy(w_hbm.at[idx_slice(s)], gs[buf], sgs[buf])

        def wait_gather(s, buf):
            pltpu.make_async_copy(w_hbm.at[idx_slice(s)], gs[buf], sgs[buf]).wait()

        def drain_out(s, buf):
            pltpu.make_async_copy(gs[buf], out_slice(s), sos[buf]).wait()

        def step(s, k, fire_next):
            wait_gather(s, k)
            if fire_next:
                nslot = (k + 3) % 4
                if isinstance(s, int):
                    if s >= 1:
                        drain_out(s - 1, nslot)
                else:

                    @pl.when(s >= 1)
                    def _():
                        drain_out(s - 1, nslot)

                fire_gather(s + 3, nslot)
            pltpu.async_copy(gs[k], out_slice(s), sos[k])

        for k in range(3):
            fire_gather(k, k)

        def quad(g4, carry):
            for k in range(4):
                s = 4 * g4 + k
                step(s, k, True)
            return carry

        lax.fori_loop(0, steps // 4 - 1, quad, 0)

        for k in range(4):
            s = steps - 4 + k
            step(s, k, s + 3 < steps)
        for k in range(4):
            s = steps - 4 + k
            drain_out(s, s % 4)

    return lookup


@jax.jit
def kernel(x, weight):
    b, h = x.shape
    xf = x.T.reshape(b * h)
    out = _make_lookup(b, h)(weight, xf)
    return jnp.transpose(out, (1, 0, 2))
